# Initial kernel scaffold; baseline (speedup 1.0000x reference)
import jax
import jax.numpy as jnp
from jax.experimental import pallas as pl


def kernel(tf_idx, emb_weight):
    raise NotImplementedError("write your pallas kernel here")


# speedup vs baseline: 2.7538x; 2.7538x over previous
import functools

import jax
import jax.numpy as jnp
from jax import lax
from jax.experimental import pallas as pl
from jax.experimental.pallas import tpu as pltpu
from jax.experimental.pallas import tpu_sc as plsc

D_MODEL = 64
N_TF = 12
NUM_ROWS_TOTAL = 16384 * 200
NUM_PAIRS = NUM_ROWS_TOTAL // 2
PAIR_W = 2 * D_MODEL
IDX_MINOR = 128
CHUNK = 256
SUB = CHUNK // IDX_MINOR


def _make_kernel():
    info = plsc.get_sparse_core_info()
    nw = info.num_cores * info.num_subcores
    rows_per_w = NUM_PAIRS // nw
    n_chunks = rows_per_w // CHUNK

    mesh = plsc.VectorSubcoreMesh(core_axis_name="c", subcore_axis_name="s")

    @functools.partial(
        pl.kernel,
        mesh=mesh,
        out_type=jax.ShapeDtypeStruct((NUM_PAIRS, PAIR_W), jnp.float32),
        scratch_types=[
            pltpu.VMEM((SUB, IDX_MINOR), jnp.int32),
            pltpu.VMEM((CHUNK, PAIR_W), jnp.float32),
            pltpu.SemaphoreType.DMA,
        ],
    )
    def emb_kernel(table_hbm, idx_hbm, out_hbm, idx_v, rows_v, sem):
        cid = lax.axis_index("c")
        sid = lax.axis_index("s")
        wid = sid * info.num_cores + cid

        idx_row0 = wid * (rows_per_w // IDX_MINOR)
        out_row0 = wid * rows_per_w

        def body(i, carry):
            pltpu.sync_copy(
                idx_hbm.at[pl.ds(idx_row0 + i * SUB, SUB)],
                idx_v,
            )
            copies = []
            for j in range(SUB):
                copies.append(pltpu.async_copy(
                    table_hbm.at[idx_v.at[j]],
                    rows_v.at[pl.ds(j * IDX_MINOR, IDX_MINOR)],
                    sem,
                ))
            for c in copies:
                c.wait()
            pltpu.sync_copy(
                rows_v,
                out_hbm.at[pl.ds(out_row0 + i * CHUNK, CHUNK)],
            )
            return carry

        lax.fori_loop(0, n_chunks, body, None)

    return emb_kernel


_EMB_KERNEL = _make_kernel()


def kernel(tf_idx, emb_weight):
    pair_table = jnp.concatenate(
        [jnp.repeat(emb_weight, N_TF, axis=0),
         jnp.tile(emb_weight, (N_TF, 1))],
        axis=1,
    )
    flat = tf_idx.reshape(NUM_PAIRS, 2)
    pidx = (flat[:, 0] * N_TF + flat[:, 1]).astype(jnp.int32)
    pidx2d = pidx.reshape(NUM_PAIRS // IDX_MINOR, IDX_MINOR)
    out = _EMB_KERNEL(pair_table, pidx2d)
    return out.reshape(tf_idx.shape[0], tf_idx.shape[1], D_MODEL)


# speedup vs baseline: 2.7624x; 1.0031x over previous
import functools

import jax
import jax.numpy as jnp
from jax import lax
from jax.experimental import pallas as pl
from jax.experimental.pallas import tpu as pltpu
from jax.experimental.pallas import tpu_sc as plsc

D_MODEL = 64
N_TF = 12
NUM_ROWS_TOTAL = 16384 * 200
NUM_PAIRS = NUM_ROWS_TOTAL // 2
PAIR_W = 2 * D_MODEL
IDX_MINOR = 128
CHUNK = 128
NSLOT = 4


def _make_kernel():
    info = plsc.get_sparse_core_info()
    nw = info.num_cores * info.num_subcores
    rows_per_w = NUM_PAIRS // nw
    n_chunks = rows_per_w // CHUNK
    n_outer = n_chunks // NSLOT

    mesh = plsc.VectorSubcoreMesh(core_axis_name="c", subcore_axis_name="s")

    @functools.partial(
        pl.kernel,
        mesh=mesh,
        out_type=jax.ShapeDtypeStruct((NUM_PAIRS, PAIR_W), jnp.float32),
        scratch_types=[
            pltpu.VMEM((n_chunks, IDX_MINOR), jnp.int32),
            pltpu.VMEM((NSLOT, CHUNK, PAIR_W), jnp.float32),
            pltpu.SemaphoreType.DMA((NSLOT,)),
            pltpu.SemaphoreType.DMA((NSLOT,)),
        ],
    )
    def emb_kernel(table_hbm, idx_hbm, out_hbm, idx_v, rows_v, gsem, osem):
        cid = lax.axis_index("c")
        sid = lax.axis_index("s")
        wid = sid * info.num_cores + cid

        idx_row0 = wid * n_chunks
        out_row0 = wid * rows_per_w

        pltpu.sync_copy(idx_hbm.at[pl.ds(idx_row0, n_chunks)], idx_v)

        def fire_gather(c, slot):
            return pltpu.async_copy(
                table_hbm.at[idx_v.at[c]], rows_v.at[slot], gsem.at[slot])

        def wait_gather(c, slot):
            pltpu.make_async_copy(
                table_hbm.at[idx_v.at[c]], rows_v.at[slot],
                gsem.at[slot]).wait()

        def fire_write(c, slot):
            return pltpu.async_copy(
                rows_v.at[slot],
                out_hbm.at[pl.ds(out_row0 + c * CHUNK, CHUNK)],
                osem.at[slot])

        def wait_write(c, slot):
            pltpu.make_async_copy(
                rows_v.at[slot],
                out_hbm.at[pl.ds(out_row0 + c * CHUNK, CHUNK)],
                osem.at[slot]).wait()

        fire_gather(0, 0)
        fire_gather(1, 1)

        def body(i, carry):
            first = i == 0
            for b in range(NSLOT):
                c = i * NSLOT + b
                nxt = (b + 2) % NSLOT
                if b < 2:
                    @pl.when(jnp.logical_not(first))
                    def _():
                        wait_write(c - 2, nxt)
                        fire_gather(c + 2, nxt)

                    @pl.when(first)
                    def _():
                        fire_gather(c + 2, nxt)
                else:
                    wait_write(c - 2, nxt)

                    @pl.when(c + 2 < n_chunks)
                    def _():
                        fire_gather(c + 2, nxt)
                wait_gather(c, b)
                fire_write(c, b)
            return carry

        lax.fori_loop(0, n_outer, body, None, unroll=False)

        wait_write(n_chunks - 2, 2)
        wait_write(n_chunks - 1, 3)

    return emb_kernel


_EMB_KERNEL = _make_kernel()


def kernel(tf_idx, emb_weight):
    pair_table = jnp.concatenate(
        [jnp.repeat(emb_weight, N_TF, axis=0),
         jnp.tile(emb_weight, (N_TF, 1))],
        axis=1,
    )
    flat = tf_idx.reshape(NUM_PAIRS, 2)
    pidx = (flat[:, 0] * N_TF + flat[:, 1]).astype(jnp.int32)
    pidx2d = pidx.reshape(NUM_PAIRS // IDX_MINOR, IDX_MINOR)
    out = _EMB_KERNEL(pair_table, pidx2d)
    return out.reshape(tf_idx.shape[0], tf_idx.shape[1], D_MODEL)


# speedup vs baseline: 41.3260x; 14.9601x over previous
import functools

import jax
import jax.numpy as jnp
from jax import lax
from jax.experimental import pallas as pl
from jax.experimental.pallas import tpu as pltpu
from jax.experimental.pallas import tpu_sc as plsc

D_MODEL = 64
N_TF = 12
TPAD = 16
N_T = 16384
N_R = 200
LANES = 16

_DNUMS = lax.GatherDimensionNumbers(
    offset_dims=(), collapsed_slice_dims=(0,), start_index_map=(0,))


def _make_kernel():
    info = plsc.get_sparse_core_info()
    nw = info.num_cores * info.num_subcores
    t_per_w = N_T // nw
    n_tb = t_per_w // LANES
    NPRE = 4

    mesh = plsc.VectorSubcoreMesh(core_axis_name="c", subcore_axis_name="s")

    @functools.partial(
        pl.kernel,
        mesh=mesh,
        compiler_params=pltpu.CompilerParams(needs_layout_passes=False),
        out_type=jax.ShapeDtypeStruct((N_R, D_MODEL, N_T), jnp.float32),
        scratch_types=[
            pltpu.VMEM((TPAD * D_MODEL,), jnp.float32),
            pltpu.VMEM((NPRE, t_per_w), jnp.int32),
            pltpu.VMEM((2, D_MODEL, t_per_w), jnp.float32),
            pltpu.SemaphoreType.DMA((NPRE,)),
            pltpu.SemaphoreType.DMA((2,)),
        ],
    )
    def emb_kernel(tab_hbm, idx_hbm, out_hbm, tab_v, idx_v, blk_v, isem, osem):
        cid = lax.axis_index("c")
        sid = lax.axis_index("s")
        wid = sid * info.num_cores + cid
        t0 = wid * t_per_w

        pltpu.sync_copy(tab_hbm, tab_v)

        def fire_idx(r, slot):
            return pltpu.async_copy(
                idx_hbm.at[r, pl.ds(t0, t_per_w)], idx_v.at[slot],
                isem.at[slot])

        def wait_idx(r, slot):
            pltpu.make_async_copy(
                idx_hbm.at[r, pl.ds(t0, t_per_w)], idx_v.at[slot],
                isem.at[slot]).wait()

        def fire_out(r, slot):
            return pltpu.async_copy(
                blk_v.at[slot], out_hbm.at[r, :, pl.ds(t0, t_per_w)],
                osem.at[slot])

        def wait_out(r, slot):
            pltpu.make_async_copy(
                blk_v.at[slot], out_hbm.at[r, :, pl.ds(t0, t_per_w)],
                osem.at[slot]).wait()

        for p in range(NPRE):
            fire_idx(p, p)

        def outer(g, carry):
            for par in range(NPRE):
                r = g * NPRE + par
                bslot = par % 2
                wait_idx(r, par)

                @pl.when(r >= 2)
                def _():
                    wait_out(r - 2, bslot)

                def cbody(c, c2):
                    tabvec = tab_v[pl.ds(c * TPAD, TPAD)]

                    @plsc.parallel_loop(0, n_tb, unroll=8)
                    def _(tb):
                        idxv = idx_v[par, pl.ds(tb * LANES, LANES)]
                        vals = lax.gather(
                            tabvec, idxv.reshape(LANES, 1), _DNUMS, (1,),
                            mode=lax.GatherScatterMode.PROMISE_IN_BOUNDS)
                        blk_v[bslot, c, pl.ds(tb * LANES, LANES)] = vals

                    return c2

                lax.fori_loop(0, D_MODEL, cbody, 0)
                fire_out(r, bslot)

                @pl.when(r + NPRE < N_R)
                def _():
                    fire_idx(r + NPRE, par)
            return carry

        lax.fori_loop(0, N_R // NPRE, outer, 0)

        wait_out(N_R - 2, 0)
        wait_out(N_R - 1, 1)

    return emb_kernel


_EMB_KERNEL = _make_kernel()


def kernel(tf_idx, emb_weight):
    tab_t = jnp.zeros((D_MODEL, TPAD), jnp.float32)
    tab_t = tab_t.at[:, :N_TF].set(emb_weight.T).reshape(TPAD * D_MODEL)
    idx_t = tf_idx.T
    out_t = _EMB_KERNEL(tab_t, idx_t)
    return jnp.transpose(out_t, (2, 0, 1))
